# Initial kernel scaffold; baseline (speedup 1.0000x reference)
import jax
import jax.numpy as jnp
from jax.experimental import pallas as pl


def kernel(feat, edge_index, in_deg, W_self0, W_neigh0, b0, W_self1, W_neigh1, b1, ln_g, ln_b):
    raise NotImplementedError("write your pallas kernel here")


# speedup vs baseline: 4.7980x; 4.7980x over previous
import functools

import jax
import jax.numpy as jnp
from jax import lax
from jax.experimental import pallas as pl
from jax.experimental.pallas import tpu as pltpu
from jax.experimental.pallas import tpu_sc as plsc

CHUNK = 128


@functools.partial(jax.jit, static_argnames=("n", "d", "nc", "ns", "nchunks"))
def _sc_aggregate(h, src3, dst3, *, n, d, nc, ns, nchunks):
    n_acc = ((n + 1 + 7) // 8) * 8
    zpart = 8 * (n_acc // (8 * ns))
    zlast = n_acc - zpart * (ns - 1)
    opart = 8 * (n // (8 * ns))
    olast = n - opart * (ns - 1)
    mesh = plsc.VectorSubcoreMesh(core_axis_name="c", subcore_axis_name="s")

    @functools.partial(
        pl.kernel,
        out_type=jax.ShapeDtypeStruct((nc, n, d), jnp.float32),
        mesh=mesh,
        scratch_types=[
            pltpu.VMEM((nchunks, CHUNK), jnp.int32),
            pltpu.VMEM((nchunks, CHUNK), jnp.int32),
            pltpu.VMEM((CHUNK, d), jnp.float32),
            pltpu.VMEM_SHARED((n_acc, d), jnp.float32),
            pltpu.SemaphoreType.DMA,
        ],
    )
    def k(h_hbm, src_hbm, dst_hbm, out_hbm, sidx_v, didx_v, rows_v, acc_sh, sem):
        cid = lax.axis_index("c")
        sid = lax.axis_index("s")
        wid = cid * ns + sid

        def zrow(i, _):
            for k8 in range(d // 16):
                rows_v[i, pl.ds(k8 * 16, 16)] = jnp.zeros((16,), jnp.float32)
            return 0

        lax.fori_loop(0, CHUNK, zrow, 0)

        def zero_rows(nrows):
            base = sid * zpart
            full, rem = nrows // CHUNK, nrows % CHUNK
            for z in range(full):
                pltpu.sync_copy(rows_v, acc_sh.at[pl.ds(base + z * CHUNK, CHUNK)])
            if rem:
                pltpu.sync_copy(rows_v.at[pl.ds(0, rem)],
                                acc_sh.at[pl.ds(base + full * CHUNK, rem)])

        pl.when(sid < ns - 1)(lambda: zero_rows(zpart))
        pl.when(sid == ns - 1)(lambda: zero_rows(zlast))
        plsc.subcore_barrier()

        pltpu.sync_copy(src_hbm.at[wid], sidx_v)
        pltpu.sync_copy(dst_hbm.at[wid], didx_v)

        def body(j, _):
            pltpu.async_copy(h_hbm.at[sidx_v.at[j]], rows_v, sem).wait()
            pltpu.sync_copy(rows_v, acc_sh.at[didx_v.at[j]], add=True)
            return 0

        lax.fori_loop(0, nchunks, body, 0)
        plsc.subcore_barrier()

        def write_rows(nrows):
            ob = sid * opart
            pltpu.sync_copy(acc_sh.at[pl.ds(ob, nrows)],
                            out_hbm.at[cid].at[pl.ds(ob, nrows)])

        pl.when(sid < ns - 1)(lambda: write_rows(opart))
        pl.when(sid == ns - 1)(lambda: write_rows(olast))

    return k(h, src3, dst3)


def _tc_body(x_ref, p0_ref, p1_ref, deg_ref, ws_ref, wn_ref, b_ref,
             g_ref, lb_ref, o_ref, *, ln_relu):
    agg = (p0_ref[...] + p1_ref[...]) / jnp.clip(deg_ref[...], 1.0, None)
    h = (
        jax.lax.dot_general(
            x_ref[...], ws_ref[...], (((1,), (0,)), ((), ())),
            preferred_element_type=jnp.float32, precision=lax.Precision.HIGHEST)
        + jax.lax.dot_general(
            agg, wn_ref[...], (((1,), (0,)), ((), ())),
            preferred_element_type=jnp.float32, precision=lax.Precision.HIGHEST)
        + b_ref[...]
    )
    if ln_relu:
        mu = jnp.mean(h, axis=-1, keepdims=True)
        var = jnp.mean(jnp.square(h - mu), axis=-1, keepdims=True)
        h = (h - mu) / jnp.sqrt(var + 1e-5) * g_ref[...] + lb_ref[...]
        h = jnp.maximum(h, 0.0)
    o_ref[...] = h


@functools.partial(jax.jit, static_argnames=("ln_relu", "br"))
def _tc_layer(x, p0, p1, deg2, ws, wn, b, g, lb, *, ln_relu, br):
    n, d = x.shape
    grid = n // br
    row_spec = pl.BlockSpec((br, d), lambda i: (i, 0))
    deg_spec = pl.BlockSpec((br, 1), lambda i: (i, 0))
    w_spec = pl.BlockSpec((d, d), lambda i: (0, 0))
    v_spec = pl.BlockSpec((1, d), lambda i: (0, 0))
    return pl.pallas_call(
        functools.partial(_tc_body, ln_relu=ln_relu),
        grid=(grid,),
        in_specs=[row_spec, row_spec, row_spec, deg_spec,
                  w_spec, w_spec, v_spec, v_spec, v_spec],
        out_specs=row_spec,
        out_shape=jax.ShapeDtypeStruct((n, d), jnp.float32),
    )(x, p0, p1, deg2, ws, wn, b, g, lb)


def kernel(feat, edge_index, in_deg, W_self0, W_neigh0, b0,
           W_self1, W_neigh1, b1, ln_g, ln_b):
    n, d = feat.shape
    e = edge_index.shape[1]
    nc, ns = 2, 16
    nw = nc * ns
    per_w = ((e + nw * CHUNK - 1) // (nw * CHUNK)) * CHUNK
    nchunks = per_w // CHUNK
    e_pad = per_w * nw

    src = edge_index[0]
    dst = edge_index[1]
    src3 = jnp.full((e_pad,), 0, jnp.int32).at[:e].set(src).reshape(nw, nchunks, CHUNK)
    dst3 = jnp.full((e_pad,), n, jnp.int32).at[:e].set(dst).reshape(nw, nchunks, CHUNK)
    deg2 = in_deg.reshape(n, 1)
    b0r, b1r = b0.reshape(1, d), b1.reshape(1, d)
    gr, lbr = ln_g.reshape(1, d), ln_b.reshape(1, d)

    br = 1000 if n % 1000 == 0 else 8 * (n // 8)

    p = _sc_aggregate(feat, src3, dst3, n=n, d=d, nc=nc, ns=ns, nchunks=nchunks)
    h1 = _tc_layer(feat, p[0], p[1], deg2, W_self0, W_neigh0, b0r, gr, lbr,
                   ln_relu=True, br=br)
    p = _sc_aggregate(h1, src3, dst3, n=n, d=d, nc=nc, ns=ns, nchunks=nchunks)
    out = _tc_layer(h1, p[0], p[1], deg2, W_self1, W_neigh1, b1r, gr, lbr,
                    ln_relu=False, br=br)
    return out
